# Initial kernel scaffold; baseline (speedup 1.0000x reference)
import jax
import jax.numpy as jnp
from jax.experimental import pallas as pl


def kernel(x, table):
    raise NotImplementedError("write your pallas kernel here")


# speedup vs baseline: 1.4374x; 1.4374x over previous
import jax
import jax.numpy as jnp
from jax import lax
from jax.experimental import pallas as pl
from jax.experimental.pallas import tpu as pltpu
from jax.experimental.pallas import tpu_sc as plsc

N_EMBED = 1000000
Z_DIM = 32
BATCH = 16384
FIELDS = 26

NC = 2
NS = 16
NW = NC * NS

B_TOTAL = BATCH * FIELDS
B_PER_W = B_TOTAL // NW
S = 128
N_STREAMS = B_PER_W // S


def _make_kernel():
    mesh = plsc.VectorSubcoreMesh(
        core_axis_name="c", subcore_axis_name="s", num_cores=NC, num_subcores=NS
    )

    @pl.kernel(
        out_type=jax.ShapeDtypeStruct((B_TOTAL, Z_DIM), jnp.float32),
        mesh=mesh,
        scratch_types=[
            pltpu.VMEM((N_STREAMS, S), jnp.int32),
            pltpu.VMEM((S, Z_DIM), jnp.float32),
            pltpu.SemaphoreType.DMA,
        ],
        compiler_params=pltpu.CompilerParams(use_tc_tiling_on_sc=False),
    )
    def gather_kernel(idx_hbm, table_hbm, out_hbm, idx_v, rows_v, sem):
        wid = lax.axis_index("s") * NC + lax.axis_index("c")
        base = wid * B_PER_W
        pltpu.sync_copy(idx_hbm.at[wid], idx_v)

        def step(j, carry):
            pltpu.async_copy(table_hbm.at[idx_v.at[j]], rows_v, sem).wait()
            pltpu.sync_copy(rows_v, out_hbm.at[pl.ds(base + j * S, S)])
            return carry

        lax.fori_loop(0, N_STREAMS, step, 0)

    return gather_kernel


def kernel(x, table):
    idx = x.reshape(NW, N_STREAMS, S)
    out = _make_kernel()(idx, table)
    return out.reshape(BATCH, FIELDS, Z_DIM)


# speedup vs baseline: 1.5645x; 1.0885x over previous
import jax
import jax.numpy as jnp
from jax import lax
from jax.experimental import pallas as pl
from jax.experimental.pallas import tpu as pltpu
from jax.experimental.pallas import tpu_sc as plsc

N_EMBED = 1000000
Z_DIM = 32
BATCH = 16384
FIELDS = 26

NC = 2
NS = 16
NW = NC * NS

B_TOTAL = BATCH * FIELDS
B_PER_W = B_TOTAL // NW
S = 128
N_STREAMS = B_PER_W // S
G = 4
GR = G * S
N_GROUPS = N_STREAMS // G


def _make_kernel():
    mesh = plsc.VectorSubcoreMesh(
        core_axis_name="c", subcore_axis_name="s", num_cores=NC, num_subcores=NS
    )

    @pl.kernel(
        out_type=jax.ShapeDtypeStruct((B_TOTAL, Z_DIM), jnp.float32),
        mesh=mesh,
        scratch_types=[
            pltpu.VMEM((N_STREAMS, S), jnp.int32),
            pltpu.VMEM((GR, Z_DIM), jnp.float32),
            pltpu.VMEM((GR, Z_DIM), jnp.float32),
            pltpu.SemaphoreType.DMA,
            pltpu.SemaphoreType.DMA,
            pltpu.SemaphoreType.DMA,
            pltpu.SemaphoreType.DMA,
        ],
        compiler_params=pltpu.CompilerParams(use_tc_tiling_on_sc=False),
    )
    def gather_kernel(idx_hbm, table_hbm, out_hbm,
                      idx_v, buf0, buf1, sg0, sg1, so0, so1):
        wid = lax.axis_index("s") * NC + lax.axis_index("c")
        base = wid * B_PER_W

        pltpu.sync_copy(idx_hbm.at[wid], idx_v)

        def fire_gather(buf, sem, g):
            for k in range(G):
                pltpu.async_copy(
                    table_hbm.at[idx_v.at[g * G + k]],
                    buf.at[pl.ds(k * S, S)], sem)

        def drain_gather(buf, sem):
            for k in range(G):
                pltpu.make_async_copy(
                    table_hbm.at[idx_v.at[0]], buf.at[pl.ds(k * S, S)], sem
                ).wait()

        def fire_out(buf, sem, g):
            pltpu.async_copy(buf, out_hbm.at[pl.ds(base + g * GR, GR)], sem)

        def drain_out(buf, sem, g):
            pltpu.make_async_copy(
                buf, out_hbm.at[pl.ds(base + g * GR, GR)], sem).wait()

        fire_gather(buf0, sg0, 0)
        fire_gather(buf1, sg1, 1)

        def body(i, carry):
            g0 = 2 * i
            g1 = 2 * i + 1
            drain_gather(buf0, sg0)
            fire_out(buf0, so0, g0)
            drain_gather(buf1, sg1)
            fire_out(buf1, so1, g1)

            @pl.when(i < N_GROUPS // 2 - 1)
            def _():
                drain_out(buf0, so0, g0)
                fire_gather(buf0, sg0, g0 + 2)
                drain_out(buf1, so1, g1)
                fire_gather(buf1, sg1, g1 + 2)

            return carry

        lax.fori_loop(0, N_GROUPS // 2, body, 0)
        drain_out(buf0, so0, N_GROUPS - 2)
        drain_out(buf1, so1, N_GROUPS - 1)

    return gather_kernel


def kernel(x, table):
    idx = x.reshape(NW, N_STREAMS, S)
    out = _make_kernel()(idx, table)
    return out.reshape(BATCH, FIELDS, Z_DIM)


# speedup vs baseline: 1.6126x; 1.0307x over previous
import jax
import jax.numpy as jnp
from jax import lax
from jax.experimental import pallas as pl
from jax.experimental.pallas import tpu as pltpu
from jax.experimental.pallas import tpu_sc as plsc

N_EMBED = 1000000
Z_DIM = 32
BATCH = 16384
FIELDS = 26

NC = 2
NS = 16
NW = NC * NS

B_TOTAL = BATCH * FIELDS
S = 128
N_UNITS = B_TOTAL // S // NW
BB = BATCH // S


def _make_kernel():
    mesh = plsc.VectorSubcoreMesh(
        core_axis_name="c", subcore_axis_name="s", num_cores=NC, num_subcores=NS
    )

    @pl.kernel(
        out_type=jax.ShapeDtypeStruct((B_TOTAL * Z_DIM,), jnp.float32),
        mesh=mesh,
        scratch_types=[
            pltpu.VMEM((N_UNITS, S), jnp.int32),
            pltpu.VMEM((S, Z_DIM), jnp.float32),
            pltpu.VMEM((S, Z_DIM), jnp.float32),
            pltpu.VMEM((Z_DIM * S,), jnp.float32),
            pltpu.VMEM((Z_DIM * S,), jnp.float32),
            pltpu.SemaphoreType.DMA,
            pltpu.SemaphoreType.DMA,
            pltpu.SemaphoreType.DMA,
            pltpu.SemaphoreType.DMA,
        ],
        compiler_params=pltpu.CompilerParams(
            use_tc_tiling_on_sc=False, needs_layout_passes=False),
    )
    def gather_kernel(idx_hbm, table_hbm, out_hbm,
                      idx_v, bufa, bufb, ta, tb, sga, sgb, soa, sob):
        wid = lax.axis_index("s") * NC + lax.axis_index("c")
        ubase = wid * N_UNITS
        pltpu.sync_copy(idx_hbm.at[wid], idx_v)

        lane128 = lax.iota(jnp.int32, 16) * 128

        def fire_gather(buf, sem, ul):
            pltpu.async_copy(table_hbm.at[idx_v.at[ul]], buf, sem)

        def drain_gather(buf, sem):
            pltpu.make_async_copy(table_hbm.at[idx_v.at[0]], buf, sem).wait()

        def transpose(buf, tbuf):
            for b in range(S):
                for j in range(Z_DIM // 16):
                    vals = buf[b, pl.ds(j * 16, 16)]
                    plsc.store_scatter(tbuf, [lane128 + (j * 2048 + b)], vals)

        def out_off(u, g):
            f = u // BB
            bb = u % BB
            return ((f * 4 + g) * BB + bb) * 1024

        def fire_out(tbuf, sem, u):
            for g in range(Z_DIM // 8):
                pltpu.async_copy(tbuf.at[pl.ds(g * 1024, 1024)],
                                 out_hbm.at[pl.ds(out_off(u, g), 1024)], sem)

        def drain_out(tbuf, sem, u):
            for g in range(Z_DIM // 8):
                pltpu.make_async_copy(
                    tbuf.at[pl.ds(g * 1024, 1024)],
                    out_hbm.at[pl.ds(out_off(u, g), 1024)], sem).wait()

        fire_gather(bufa, sga, 0)
        fire_gather(bufb, sgb, 1)

        def body(i, carry):
            la = 2 * i
            lb = la + 1
            ua = ubase + la
            ub = ua + 1

            drain_gather(bufa, sga)

            @pl.when(i > 0)
            def _():
                drain_out(ta, soa, ua - 2)

            transpose(bufa, ta)
            fire_out(ta, soa, ua)

            @pl.when(i < N_UNITS // 2 - 1)
            def _():
                fire_gather(bufa, sga, la + 2)

            drain_gather(bufb, sgb)

            @pl.when(i > 0)
            def _():
                drain_out(tb, sob, ub - 2)

            transpose(bufb, tb)
            fire_out(tb, sob, ub)

            @pl.when(i < N_UNITS // 2 - 1)
            def _():
                fire_gather(bufb, sgb, lb + 2)

            return carry

        lax.fori_loop(0, N_UNITS // 2, body, 0)
        drain_out(ta, soa, ubase + N_UNITS - 2)
        drain_out(tb, sob, ubase + N_UNITS - 1)

    return gather_kernel


def kernel(x, table):
    idx = x.T.reshape(NW, N_UNITS, S)
    flat = _make_kernel()(idx, table)
    out5 = flat.reshape(FIELDS, Z_DIM // 8, BB, 8, S)
    return out5.transpose(2, 4, 0, 1, 3).reshape(BATCH, FIELDS, Z_DIM)


# speedup vs baseline: 1.6144x; 1.0011x over previous
import jax
import jax.numpy as jnp
from jax import lax
from jax.experimental import pallas as pl
from jax.experimental.pallas import tpu as pltpu
from jax.experimental.pallas import tpu_sc as plsc

N_EMBED = 1000000
Z_DIM = 32
BATCH = 16384
FIELDS = 26

NC = 2
NS = 16
NW = NC * NS

B_TOTAL = BATCH * FIELDS
S = 128
N_UNITS = B_TOTAL // S // NW
BB = BATCH // S
G = 4
N_GROUPS = N_UNITS // G
GPF = BB // G
GW = Z_DIM * S


def _make_kernel():
    mesh = plsc.VectorSubcoreMesh(
        core_axis_name="c", subcore_axis_name="s", num_cores=NC, num_subcores=NS
    )

    @pl.kernel(
        out_type=jax.ShapeDtypeStruct((B_TOTAL * Z_DIM,), jnp.float32),
        mesh=mesh,
        scratch_types=[
            pltpu.VMEM((N_UNITS, S), jnp.int32),
            pltpu.VMEM((G * S, Z_DIM), jnp.float32),
            pltpu.VMEM((G * S, Z_DIM), jnp.float32),
            pltpu.VMEM((G * GW,), jnp.float32),
            pltpu.VMEM((G * GW,), jnp.float32),
            pltpu.SemaphoreType.DMA,
            pltpu.SemaphoreType.DMA,
            pltpu.SemaphoreType.DMA,
            pltpu.SemaphoreType.DMA,
        ],
        compiler_params=pltpu.CompilerParams(
            use_tc_tiling_on_sc=False, needs_layout_passes=False),
    )
    def gather_kernel(idx_hbm, table_hbm, out_hbm,
                      idx_v, bufa, bufb, ta, tb, sga, sgb, soa, sob):
        wid = lax.axis_index("s") * NC + lax.axis_index("c")
        gbase = wid * N_GROUPS
        pltpu.sync_copy(idx_hbm.at[wid], idx_v)

        lane = lax.iota(jnp.int32, 16)
        vb0 = (lane // 8) * (G * 1024) + (lane % 8) * 128
        vb1 = (2 + lane // 8) * (G * 1024) + (lane % 8) * 128

        def fire_gather(buf, sem, gl):
            for k in range(G):
                pltpu.async_copy(table_hbm.at[idx_v.at[gl * G + k]],
                                 buf.at[pl.ds(k * S, S)], sem)

        def drain_gather(buf, sem):
            for k in range(G):
                pltpu.make_async_copy(table_hbm.at[idx_v.at[0]],
                                      buf.at[pl.ds(k * S, S)], sem).wait()

        def transpose(buf, tbuf):
            def step(it, carry):
                b = it * 2
                for b2 in range(2):
                    for k in range(G):
                        row = k * S + b + b2
                        off = k * 1024 + b + b2
                        v0 = buf[row, pl.ds(0, 16)]
                        plsc.store_scatter(tbuf, [vb0 + off], v0)
                        v1 = buf[row, pl.ds(16, 16)]
                        plsc.store_scatter(tbuf, [vb1 + off], v1)
                return carry

            lax.fori_loop(0, S // 2, step, 0)

        def out_off(gg, g):
            f = gg // GPF
            bb0 = (gg % GPF) * G
            return ((f * 4 + g) * BB + bb0) * 1024

        def fire_out(tbuf, sem, gg):
            for g in range(Z_DIM // 8):
                pltpu.async_copy(tbuf.at[pl.ds(g * G * 1024, G * 1024)],
                                 out_hbm.at[pl.ds(out_off(gg, g), G * 1024)],
                                 sem)

        def drain_out(tbuf, sem, gg):
            for g in range(Z_DIM // 8):
                pltpu.make_async_copy(
                    tbuf.at[pl.ds(g * G * 1024, G * 1024)],
                    out_hbm.at[pl.ds(out_off(gg, g), G * 1024)], sem).wait()

        fire_gather(bufa, sga, 0)
        fire_gather(bufb, sgb, 1)

        def body(i, carry):
            la = 2 * i
            lb = la + 1
            ga = gbase + la
            gb = ga + 1

            drain_gather(bufa, sga)

            @pl.when(i > 0)
            def _():
                drain_out(ta, soa, ga - 2)

            transpose(bufa, ta)
            fire_out(ta, soa, ga)

            @pl.when(i < N_GROUPS // 2 - 1)
            def _():
                fire_gather(bufa, sga, la + 2)

            drain_gather(bufb, sgb)

            @pl.when(i > 0)
            def _():
                drain_out(tb, sob, gb - 2)

            transpose(bufb, tb)
            fire_out(tb, sob, gb)

            @pl.when(i < N_GROUPS // 2 - 1)
            def _():
                fire_gather(bufb, sgb, lb + 2)

            return carry

        lax.fori_loop(0, N_GROUPS // 2, body, 0)
        drain_out(ta, soa, gbase + N_GROUPS - 2)
        drain_out(tb, sob, gbase + N_GROUPS - 1)

    return gather_kernel


def kernel(x, table):
    idx = x.T.reshape(NW, N_UNITS, S)
    flat = _make_kernel()(idx, table)
    out5 = flat.reshape(FIELDS, Z_DIM // 8, BB, 8, S)
    return out5.transpose(2, 4, 0, 1, 3).reshape(BATCH, FIELDS, Z_DIM)


# speedup vs baseline: 1.6367x; 1.0138x over previous
import jax
import jax.numpy as jnp
from jax import lax
from jax.experimental import pallas as pl
from jax.experimental.pallas import tpu as pltpu
from jax.experimental.pallas import tpu_sc as plsc

N_EMBED = 1000000
Z_DIM = 32
BATCH = 16384
FIELDS = 26

NC = 2
NS = 16
NW = NC * NS

B_TOTAL = BATCH * FIELDS
S = 128
N_UNITS = B_TOTAL // S // NW
BB = BATCH // S
G = 2
N_GROUPS = N_UNITS // G
GPF = BB // G
GW = Z_DIM * S


def _make_kernel():
    mesh = plsc.VectorSubcoreMesh(
        core_axis_name="c", subcore_axis_name="s", num_cores=NC, num_subcores=NS
    )

    @pl.kernel(
        out_type=jax.ShapeDtypeStruct((B_TOTAL * Z_DIM,), jnp.float32),
        mesh=mesh,
        scratch_types=[
            pltpu.VMEM((N_UNITS, S), jnp.int32),
            pltpu.VMEM((G * S, 128), jnp.float32),
            pltpu.VMEM((G * S, 128), jnp.float32),
            pltpu.VMEM((G * GW,), jnp.float32),
            pltpu.VMEM((G * GW,), jnp.float32),
            pltpu.SemaphoreType.DMA,
            pltpu.SemaphoreType.DMA,
            pltpu.SemaphoreType.DMA,
            pltpu.SemaphoreType.DMA,
        ],
        compiler_params=pltpu.CompilerParams(
            use_tc_tiling_on_sc=False, needs_layout_passes=False),
    )
    def gather_kernel(idx_hbm, table_hbm, out_hbm,
                      idx_v, bufa, bufb, ta, tb, sga, sgb, soa, sob):
        wid = lax.axis_index("s") * NC + lax.axis_index("c")
        gbase = wid * N_GROUPS
        pltpu.sync_copy(idx_hbm.at[wid], idx_v)

        lane = lax.iota(jnp.int32, 16)
        vb0 = (lane // 8) * (G * 1024) + (lane % 8) * 128
        vb1 = (2 + lane // 8) * (G * 1024) + (lane % 8) * 128

        def fire_gather(buf, sem, gl):
            for k in range(G):
                pltpu.async_copy(table_hbm.at[idx_v.at[gl * G + k]],
                                 buf.at[pl.ds(k * S, S)], sem)

        def drain_gather(buf, sem):
            for k in range(G):
                pltpu.make_async_copy(table_hbm.at[idx_v.at[0]],
                                      buf.at[pl.ds(k * S, S)], sem).wait()

        def transpose(buf, tbuf):
            def step(it, carry):
                b = it * 2
                for b2 in range(2):
                    for k in range(G):
                        row = k * S + b + b2
                        off = k * 1024 + b + b2
                        v0 = buf[row, pl.ds(0, 16)]
                        plsc.store_scatter(tbuf, [vb0 + off], v0)
                        v1 = buf[row, pl.ds(16, 16)]
                        plsc.store_scatter(tbuf, [vb1 + off], v1)
                return carry

            lax.fori_loop(0, S // 2, step, 0)

        def out_off(gg, g):
            f = gg // GPF
            bb0 = (gg % GPF) * G
            return ((f * 4 + g) * BB + bb0) * 1024

        def fire_out(tbuf, sem, gg):
            for g in range(Z_DIM // 8):
                pltpu.async_copy(tbuf.at[pl.ds(g * G * 1024, G * 1024)],
                                 out_hbm.at[pl.ds(out_off(gg, g), G * 1024)],
                                 sem)

        def drain_out(tbuf, sem, gg):
            for g in range(Z_DIM // 8):
                pltpu.make_async_copy(
                    tbuf.at[pl.ds(g * G * 1024, G * 1024)],
                    out_hbm.at[pl.ds(out_off(gg, g), G * 1024)], sem).wait()

        fire_gather(bufa, sga, 0)
        fire_gather(bufb, sgb, 1)

        def body(i, carry):
            la = 2 * i
            lb = la + 1
            ga = gbase + la
            gb = ga + 1

            drain_gather(bufa, sga)

            @pl.when(i > 0)
            def _():
                drain_out(ta, soa, ga - 2)

            transpose(bufa, ta)
            fire_out(ta, soa, ga)

            @pl.when(i < N_GROUPS // 2 - 1)
            def _():
                fire_gather(bufa, sga, la + 2)

            drain_gather(bufb, sgb)

            @pl.when(i > 0)
            def _():
                drain_out(tb, sob, gb - 2)

            transpose(bufb, tb)
            fire_out(tb, sob, gb)

            @pl.when(i < N_GROUPS // 2 - 1)
            def _():
                fire_gather(bufb, sgb, lb + 2)

            return carry

        lax.fori_loop(0, N_GROUPS // 2, body, 0)
        drain_out(ta, soa, gbase + N_GROUPS - 2)
        drain_out(tb, sob, gbase + N_GROUPS - 1)

    return gather_kernel


def kernel(x, table):
    idx = x.T.reshape(NW, N_UNITS, S)
    tpad = jnp.pad(table, ((0, 0), (0, 128 - Z_DIM)))
    flat = _make_kernel()(idx, tpad)
    out5 = flat.reshape(FIELDS, Z_DIM // 8, BB, 8, S)
    return out5.transpose(2, 4, 0, 1, 3).reshape(BATCH, FIELDS, Z_DIM)
